# baseline (speedup 1.0000x reference)
import jax
import jax.numpy as jnp
from jax.experimental import pallas as pl

_DIM = 1024
_SCALE = _DIM ** (-0.5)
_BLOCK_ROWS = 1024


def _scale_kernel(emb_ref, out_ref):
    out_ref[...] = emb_ref[...] * _SCALE


def _half(emb_half):
    rows = emb_half.shape[0]
    return pl.pallas_call(
        _scale_kernel,
        grid=(rows // _BLOCK_ROWS,),
        in_specs=[pl.BlockSpec((_BLOCK_ROWS, _DIM), lambda i: (i, 0))],
        out_specs=pl.BlockSpec((_BLOCK_ROWS, _DIM), lambda i: (i, 0)),
        out_shape=jax.ShapeDtypeStruct((rows, _DIM), emb_half.dtype),
    )(emb_half)


def kernel(x, emb):
    rows = emb.shape[0]
    half = rows // 2
    return jnp.concatenate([_half(emb[:half]), _half(emb[half:])], axis=0)
